# Initial kernel scaffold; baseline (speedup 1.0000x reference)
import jax
import jax.numpy as jnp
from jax.experimental import pallas as pl


def kernel(B, table):
    raise NotImplementedError("write your pallas kernel here")


# speedup vs baseline: 2.8329x; 2.8329x over previous
import functools

import jax
import jax.numpy as jnp
from jax import lax
from jax.experimental import pallas as pl
from jax.experimental.pallas import tpu as pltpu
from jax.experimental.pallas import tpu_sc as plsc

N, H, W = 16, 56, 56
HW = H * W
M = 32
KBITS = 10
D = 64
TS = 2 ** KBITS
CHUNK = 448
NCHUNK = HW // CHUNK
NGROUP = CHUNK // 16
NC, NS = 2, 16
L = 16


def _full(v, dtype=jnp.float32):
    return jnp.full((L,), v, dtype=dtype)


def _sc_partial(Br, Tr):
    mesh = plsc.VectorSubcoreMesh(
        core_axis_name="c", subcore_axis_name="s", num_cores=NC, num_subcores=NS
    )

    @functools.partial(
        pl.kernel,
        out_type=jax.ShapeDtypeStruct((NC, N, D, HW), jnp.float32),
        mesh=mesh,
        scratch_types=[
            pltpu.VMEM((TS * D,), jnp.float32),
            pltpu.VMEM((KBITS, CHUNK), jnp.float32),
            pltpu.VMEM((D, CHUNK), jnp.float32),
            pltpu.VMEM((D,), jnp.int32),
            pltpu.VMEM_SHARED((D, CHUNK), jnp.float32),
        ],
        compiler_params=pltpu.CompilerParams(
            use_tc_tiling_on_sc=False, needs_layout_passes=False
        ),
    )
    def k(b_hbm, t_hbm, out_hbm, tab_v, bits_v, acc_v, rows_v, acc_sh):
        c = lax.axis_index("c")
        s = lax.axis_index("s")
        f = c * NS + s

        pltpu.sync_copy(t_hbm.at[f], tab_v)
        for i in range(D // L):
            rows_v[pl.ds(i * L, L)] = lax.iota(jnp.int32, L) + i * L

        def group_body(g, carry):
            base = g * L
            T = [bits_v[kk, pl.ds(base, L)] for kk in range(KBITS)]
            half = _full(0.5)
            word = _full(0, jnp.int32)
            for kk in range(KBITS):
                bit = T[kk] >= half
                word = word | jnp.where(bit, _full(1 << (KBITS - 1 - kk), jnp.int32),
                                        _full(0, jnp.int32))
            BA = [jnp.abs(T[kk] - half) for kk in range(KBITS)]
            bv, bi, bt = BA[0], _full(0, jnp.int32), T[0]
            for kk in range(1, KBITS):
                lt = BA[kk] < bv
                bv = jnp.where(lt, BA[kk], bv)
                bi = jnp.where(lt, _full(kk, jnp.int32), bi)
                bt = jnp.where(lt, T[kk], bt)
            one = _full(1.0)
            zero = _full(0.0)
            v0 = BA[0] + jnp.where(bi == _full(0, jnp.int32), one, zero)
            b2v, b2i, b2t = v0, _full(0, jnp.int32), T[0]
            for kk in range(1, KBITS):
                vk = BA[kk] + jnp.where(bi == _full(kk, jnp.int32), one, zero)
                lt = vk < b2v
                b2v = jnp.where(lt, vk, b2v)
                b2i = jnp.where(lt, _full(kk, jnp.int32), b2i)
                b2t = jnp.where(lt, T[kk], b2t)
            onei = _full(1, jnp.int32)
            m0 = lax.shift_left(onei, _full(KBITS - 1, jnp.int32) - bi)
            m1 = lax.shift_left(onei, _full(KBITS - 1, jnp.int32) - b2i)
            nm0 = ~m0
            nm1 = ~m1
            w00 = (word & nm0) & nm1
            w10 = (word | m0) & nm1
            w01 = (word & nm0) | m1
            w11 = (word | m0) | m1
            t0, t1 = bt, b2t
            at00 = (one - t0) * (one - t1)
            at10 = t0 * (one - t1)
            at01 = (one - t0) * t1
            at11 = t0 * t1
            sixty4 = _full(D, jnp.int32)
            ii = [w00 * sixty4, w10 * sixty4, w01 * sixty4, w11 * sixty4]
            for j in range(D):
                g0 = plsc.load_gather(tab_v, [ii[0]])
                g1 = plsc.load_gather(tab_v, [ii[1]])
                g2 = plsc.load_gather(tab_v, [ii[2]])
                g3 = plsc.load_gather(tab_v, [ii[3]])
                tmp = at00 * g0 + at10 * g1 + at01 * g2 + at11 * g3
                acc_v[j, pl.ds(base, L)] = tmp
                if j + 1 < D:
                    ii = [x + onei for x in ii]
            return carry

        def chunk_body(n, ch):
            pltpu.sync_copy(b_hbm.at[n, f, :, pl.ds(ch * CHUNK, CHUNK)], bits_v)
            lax.fori_loop(0, NGROUP, group_body, 0)

            @pl.when(s == 0)
            def _():
                pltpu.sync_copy(acc_v, acc_sh)

            plsc.subcore_barrier()

            @pl.when(s != 0)
            def _():
                pltpu.sync_copy(acc_v, acc_sh.at[rows_v], add=True)

            plsc.subcore_barrier()

            @pl.when(s == 0)
            def _():
                pltpu.sync_copy(acc_sh, out_hbm.at[c, n, :, pl.ds(ch * CHUNK, CHUNK)])

            plsc.subcore_barrier()

        def n_body(n, carry):
            def ch_body(ch, carry2):
                chunk_body(n, ch)
                return carry2
            lax.fori_loop(0, NCHUNK, ch_body, 0)
            return carry

        lax.fori_loop(0, N, n_body, 0)

    return k(Br, Tr)


def _tc_combine(p0, p1):
    def body(a_ref, b_ref, o_ref):
        o_ref[...] = a_ref[...] + b_ref[...]

    spec = pl.BlockSpec((1, D, HW), lambda n: (n, 0, 0))
    return pl.pallas_call(
        body,
        grid=(N,),
        in_specs=[spec, spec],
        out_specs=spec,
        out_shape=jax.ShapeDtypeStruct((N, D, HW), jnp.float32),
    )(p0, p1)


def kernel(B, table):
    Br = B.reshape(N, M, KBITS, HW)
    Tr = table.reshape(M, TS * D)
    partial = _sc_partial(Br, Tr)
    out = _tc_combine(partial[0], partial[1])
    return out.reshape(N, D, H, W)


# speedup vs baseline: 27.3051x; 9.6387x over previous
import functools

import jax
import jax.numpy as jnp
from jax import lax
from jax.experimental import pallas as pl
from jax.experimental.pallas import tpu as pltpu
from jax.experimental.pallas import tpu_sc as plsc

N, H, W = 16, 56, 56
HW = H * W
M = 32
KBITS = 10
D = 64
TS = 2 ** KBITS
CHUNK = 448
NCHUNK = HW // CHUNK
NGROUP = CHUNK // 16
NC, NS = 2, 16
L = 16


def _full(v, dtype=jnp.float32):
    return jnp.full((L,), v, dtype=dtype)


def _sc_partial(Br, Tr):
    mesh = plsc.VectorSubcoreMesh(
        core_axis_name="c", subcore_axis_name="s", num_cores=NC, num_subcores=NS
    )

    @functools.partial(
        pl.kernel,
        out_type=jax.ShapeDtypeStruct((NC, N, D, HW), jnp.float32),
        mesh=mesh,
        scratch_types=[
            pltpu.VMEM((TS * D,), jnp.float32),
            pltpu.VMEM((KBITS, CHUNK), jnp.float32),
            pltpu.VMEM((D, CHUNK), jnp.float32),
            pltpu.VMEM((D,), jnp.int32),
            pltpu.VMEM_SHARED((D, CHUNK), jnp.float32),
        ],
        compiler_params=pltpu.CompilerParams(
            use_tc_tiling_on_sc=False, needs_layout_passes=False
        ),
    )
    def k(b_hbm, t_hbm, out_hbm, tab_v, bits_v, acc_v, rows_v, acc_sh):
        c = lax.axis_index("c")
        s = lax.axis_index("s")
        f = c * NS + s

        pltpu.sync_copy(t_hbm.at[f], tab_v)
        for i in range(D // L):
            rows_v[pl.ds(i * L, L)] = lax.iota(jnp.int32, L) + i * L

        def group_body(g, carry):
            base = g * L
            T = [bits_v[kk, pl.ds(base, L)] for kk in range(KBITS)]
            half = _full(0.5)
            word = _full(0, jnp.int32)
            for kk in range(KBITS):
                bit = T[kk] >= half
                word = word | jnp.where(bit, _full(1 << (KBITS - 1 - kk), jnp.int32),
                                        _full(0, jnp.int32))
            BA = [jnp.abs(T[kk] - half) for kk in range(KBITS)]
            bv, bi, bt = BA[0], _full(0, jnp.int32), T[0]
            for kk in range(1, KBITS):
                lt = BA[kk] < bv
                bv = jnp.where(lt, BA[kk], bv)
                bi = jnp.where(lt, _full(kk, jnp.int32), bi)
                bt = jnp.where(lt, T[kk], bt)
            one = _full(1.0)
            zero = _full(0.0)
            v0 = BA[0] + jnp.where(bi == _full(0, jnp.int32), one, zero)
            b2v, b2i, b2t = v0, _full(0, jnp.int32), T[0]
            for kk in range(1, KBITS):
                vk = BA[kk] + jnp.where(bi == _full(kk, jnp.int32), one, zero)
                lt = vk < b2v
                b2v = jnp.where(lt, vk, b2v)
                b2i = jnp.where(lt, _full(kk, jnp.int32), b2i)
                b2t = jnp.where(lt, T[kk], b2t)
            onei = _full(1, jnp.int32)
            m0 = lax.shift_left(onei, _full(KBITS - 1, jnp.int32) - bi)
            m1 = lax.shift_left(onei, _full(KBITS - 1, jnp.int32) - b2i)
            nm0 = ~m0
            nm1 = ~m1
            w00 = (word & nm0) & nm1
            w10 = (word | m0) & nm1
            w01 = (word & nm0) | m1
            w11 = (word | m0) | m1
            t0, t1 = bt, b2t
            at00 = (one - t0) * (one - t1)
            at10 = t0 * (one - t1)
            at01 = (one - t0) * t1
            at11 = t0 * t1
            tsc = _full(TS, jnp.int32)
            ii = [w00, w10, w01, w11]
            for j in range(D):
                g0 = plsc.load_gather(tab_v, [ii[0]])
                g1 = plsc.load_gather(tab_v, [ii[1]])
                g2 = plsc.load_gather(tab_v, [ii[2]])
                g3 = plsc.load_gather(tab_v, [ii[3]])
                tmp = (at00 * g0 + at10 * g1) + (at01 * g2 + at11 * g3)
                acc_v[j, pl.ds(base, L)] = tmp
                if j + 1 < D:
                    ii = [x + tsc for x in ii]
            return carry

        def chunk_body(n, ch):
            pltpu.sync_copy(b_hbm.at[n, f, :, pl.ds(ch * CHUNK, CHUNK)], bits_v)
            lax.fori_loop(0, NGROUP, group_body, 0)

            @pl.when(s == 0)
            def _():
                pltpu.sync_copy(acc_v, acc_sh)

            plsc.subcore_barrier()

            @pl.when(s != 0)
            def _():
                pltpu.sync_copy(acc_v, acc_sh.at[rows_v], add=True)

            plsc.subcore_barrier()

            @pl.when(s == 0)
            def _():
                pltpu.sync_copy(acc_sh, out_hbm.at[c, n, :, pl.ds(ch * CHUNK, CHUNK)])

            plsc.subcore_barrier()

        def n_body(n, carry):
            def ch_body(ch, carry2):
                chunk_body(n, ch)
                return carry2
            lax.fori_loop(0, NCHUNK, ch_body, 0)
            return carry

        lax.fori_loop(0, N, n_body, 0)

    return k(Br, Tr)


def _tc_combine(p0, p1):
    def body(a_ref, b_ref, o_ref):
        o_ref[...] = a_ref[...] + b_ref[...]

    spec = pl.BlockSpec((1, D, HW), lambda n: (n, 0, 0))
    return pl.pallas_call(
        body,
        grid=(N,),
        in_specs=[spec, spec],
        out_specs=spec,
        out_shape=jax.ShapeDtypeStruct((N, D, HW), jnp.float32),
    )(p0, p1)


def kernel(B, table):
    Br = B.reshape(N, M, KBITS, HW)
    Tr = jnp.swapaxes(table, 1, 2).reshape(M, D * TS)
    partial = _sc_partial(Br, Tr)
    out = _tc_combine(partial[0], partial[1])
    return out.reshape(N, D, H, W)
